# baseline (speedup 1.0000x reference)
import jax
import jax.numpy as jnp
from jax import lax
from jax.experimental import pallas as pl
from jax.experimental.pallas import tpu as pltpu
from jax.experimental.pallas import tpu_sc as plsc

VOCAB = 1000000
EMBED = 64
BATCH = 4096
SEQLEN = 50

NC = 2
NS = 16
LANES = 16
NW = NC * NS
BB = BATCH // NW
NGR = BB // LANES
ZROW = BB

NSTRIP = VOCAB // 128
SW = 256
NPAIR = NSTRIP // 2
PAIRS_MAIN = NPAIR - (NPAIR % NW)
PER_W_PAIRS = PAIRS_MAIN // NW
TAIL_PAIRS = NPAIR % NW
TAIL_V = VOCAB - NSTRIP * 128


def _iota16():
    return lax.iota(jnp.int32, LANES)


def _full16(v):
    return jnp.full((LANES,), v, jnp.int32)


def _transpose_strip(sb, tb, ngroups):
    rots = [
        jnp.bitwise_and(_iota16() + _full16(k), _full16(LANES - 1))
        for k in range(LANES)
    ]

    @plsc.parallel_loop(0, ngroups, unroll=1)
    def grp(vg):
        vlv = vg * LANES + _iota16()
        trow = lax.shift_right_logical(vlv, 1)
        tcolb = lax.shift_left(jnp.bitwise_and(vlv, _full16(1)), _full16(6))
        for fb in range(EMBED // LANES):
            fbase = _full16(fb * LANES)
            for k in range(LANES):
                fvec = fbase + rots[k]
                val = plsc.load_gather(sb, [fvec, vlv])
                plsc.store_scatter(tb, [trow, tcolb + fvec], val)


def _format_body(tT_hbm, tailp_hbm, out_hbm,
                 sb0, sb1, tb0, tb1, gs0, gs1, os0, os1):
    wid = lax.axis_index("s") * NC + lax.axis_index("c")
    bufs = ((sb0, tb0, gs0, os0), (sb1, tb1, gs1, os1))

    def pair_of(j):
        return wid + j * NW

    def fire_in(j, sb, gs):
        col = pl.multiple_of(pair_of(j) * SW, SW)
        pltpu.async_copy(tT_hbm.at[:, pl.ds(col, SW)], sb, gs)

    def drain_in(sb, gs):
        pltpu.make_async_copy(
            tT_hbm.at[:, pl.ds(0, SW)], sb, gs
        ).wait()

    def fire_out(j, tb, os):
        row = pl.multiple_of(pair_of(j) * (SW // 2), SW // 2)
        pltpu.async_copy(tb, out_hbm.at[pl.ds(row, SW // 2)], os)

    def drain_out(tb, os):
        pltpu.make_async_copy(
            tb, out_hbm.at[pl.ds(0, SW // 2)], os
        ).wait()

    for par in (0, 1):
        sb, tb, gs, os = bufs[par]
        fire_in(jnp.int32(par), sb, gs)
    for par in (0, 1):
        sb, tb, gs, os = bufs[par]
        drain_in(sb, gs)
        _transpose_strip(sb, tb, SW // LANES)
        fire_out(jnp.int32(par), tb, os)
        fire_in(jnp.int32(par + 2), sb, gs)

    def loop_body(i, carry):
        for par in (0, 1):
            sb, tb, gs, os = bufs[par]
            j = 2 * i + par
            drain_in(sb, gs)
            drain_out(tb, os)
            _transpose_strip(sb, tb, SW // LANES)
            fire_out(j, tb, os)
            fire_in(jnp.minimum(j + 2, PER_W_PAIRS - 1), sb, gs)
        return carry

    lax.fori_loop(1, PER_W_PAIRS // 2, loop_body, 0)

    for par in (0, 1):
        sb, tb, gs, os = bufs[par]
        drain_in(sb, gs)
        drain_out(tb, os)

    @pl.when(wid < TAIL_PAIRS)
    def _tail_full():
        p = PAIRS_MAIN + wid
        col = pl.multiple_of(p * SW, SW)
        pltpu.sync_copy(tT_hbm.at[:, pl.ds(col, SW)], sb0)
        _transpose_strip(sb0, tb0, SW // LANES)
        pltpu.sync_copy(tb0, out_hbm.at[pl.ds(p * (SW // 2), SW // 2)])

    @pl.when(wid == TAIL_PAIRS)
    def _tail_half():
        pltpu.sync_copy(tailp_hbm, tb0.at[pl.ds(0, TAIL_V // 2)])
        pltpu.sync_copy(
            tb0.at[pl.ds(0, TAIL_V // 2)],
            out_hbm.at[pl.ds(NSTRIP * 64, TAIL_V // 2)],
        )


def _gather_body(xT_hbm, tab_hbm, out_hbm,
                 iv_v, idx2_0, idx2_1, rz0, rz1, tb0, tb1,
                 gs0, gs1, os0, os1):
    wid = lax.axis_index("s") * NC + lax.axis_index("c")
    bcol = pl.multiple_of(wid * BB, BB)

    pltpu.sync_copy(xT_hbm.at[:, pl.ds(bcol, BB)], iv_v)

    for c in range(NGR):
        rz0[ZROW, pl.ds(c * LANES, LANES)] = jnp.zeros((LANES,), jnp.float32)
        rz1[ZROW, pl.ds(c * LANES, LANES)] = jnp.zeros((LANES,), jnp.float32)

    def prep(l, idx2_v):
        for g in range(NGR):
            ivg = plsc.load_gather(
                iv_v, [_full16(l), g * LANES + _iota16()]
            )
            idx2_v[pl.ds(g * LANES, LANES)] = lax.shift_right_logical(ivg, 1)

    def fire(idx2_v, rz, sem):
        pltpu.async_copy(tab_hbm.at[idx2_v], rz.at[pl.ds(0, BB)], sem)

    def drain_g(rz, sem):
        pltpu.make_async_copy(
            tab_hbm.at[pl.ds(0, BB)], rz.at[pl.ds(0, BB)], sem
        ).wait()

    def transpose(l, rz, tb):
        def bgrp_body(bg, carry):
            ivg = plsc.load_gather(
                iv_v, [_full16(l), bg * LANES + _iota16()]
            )
            half = lax.shift_left(
                jnp.bitwise_and(ivg, _full16(1)), _full16(6)
            )
            rowv = jnp.where(
                ivg == 0, _full16(ZROW), bg * LANES + _iota16()
            )
            colv = bg * LANES + _iota16()
            for fb in range(EMBED // LANES):
                fbase = _full16(fb * LANES)

                @plsc.parallel_loop(0, LANES, unroll=8)
                def k_body(k):
                    fvec = fbase + jnp.bitwise_and(
                        _iota16() + _full16(k), _full16(LANES - 1)
                    )
                    val = plsc.load_gather(rz, [rowv, half + fvec])
                    plsc.store_scatter(tb, [fvec, colv], val)

            return carry

        lax.fori_loop(0, NGR, bgrp_body, 0)

    def outdma(l, tb, sem):
        pltpu.async_copy(tb, out_hbm.at[l, :, pl.ds(bcol, BB)], sem)

    def drain_o(tb, sem):
        pltpu.make_async_copy(
            tb, out_hbm.at[0, :, pl.ds(bcol, BB)], sem
        ).wait()

    bufs = ((idx2_0, rz0, tb0, gs0, os0), (idx2_1, rz1, tb1, gs1, os1))

    for par in (0, 1):
        idx2_v, rz, tb, gs, os = bufs[par]
        prep(jnp.int32(par), idx2_v)
        fire(idx2_v, rz, gs)

    for par in (0, 1):
        idx2_v, rz, tb, gs, os = bufs[par]
        l = jnp.int32(par)
        drain_g(rz, gs)
        transpose(l, rz, tb)
        outdma(l, tb, os)
        prep(l + 2, idx2_v)
        fire(idx2_v, rz, gs)

    def loop_body(i, carry):
        for par in (0, 1):
            idx2_v, rz, tb, gs, os = bufs[par]
            l = 2 * i + par
            drain_g(rz, gs)
            drain_o(tb, os)
            transpose(l, rz, tb)
            outdma(l, tb, os)
            prep(jnp.minimum(l + 2, SEQLEN - 1), idx2_v)
            fire(idx2_v, rz, gs)
        return carry

    lax.fori_loop(1, SEQLEN // 2, loop_body, 0)

    for par in (0, 1):
        idx2_v, rz, tb, gs, os = bufs[par]
        drain_g(rz, gs)
        drain_o(tb, os)


@jax.jit
def _sc_embed(xT, tT, tailpair):
    fmt = pl.kernel(
        _format_body,
        out_type=jax.ShapeDtypeStruct((VOCAB // 2, 2 * EMBED), jnp.float32),
        mesh=plsc.VectorSubcoreMesh(
            core_axis_name="c", subcore_axis_name="s"
        ),
        scratch_types=[
            pltpu.VMEM((EMBED, SW), jnp.float32),
            pltpu.VMEM((EMBED, SW), jnp.float32),
            pltpu.VMEM((SW // 2, 128), jnp.float32),
            pltpu.VMEM((SW // 2, 128), jnp.float32),
            pltpu.SemaphoreType.DMA,
            pltpu.SemaphoreType.DMA,
            pltpu.SemaphoreType.DMA,
            pltpu.SemaphoreType.DMA,
        ],
        compiler_params=pltpu.CompilerParams(
            use_tc_tiling_on_sc=True, needs_layout_passes=False
        ),
    )
    tab2 = fmt(tT, tailpair)

    gat = pl.kernel(
        _gather_body,
        out_type=jax.ShapeDtypeStruct((SEQLEN, EMBED, BATCH), jnp.float32),
        mesh=plsc.VectorSubcoreMesh(
            core_axis_name="c", subcore_axis_name="s"
        ),
        scratch_types=[
            pltpu.VMEM((SEQLEN, BB), jnp.int32),
            pltpu.VMEM((BB,), jnp.int32),
            pltpu.VMEM((BB,), jnp.int32),
            pltpu.VMEM((BB + 8, BB), jnp.float32),
            pltpu.VMEM((BB + 8, BB), jnp.float32),
            pltpu.VMEM((EMBED, BB), jnp.float32),
            pltpu.VMEM((EMBED, BB), jnp.float32),
            pltpu.SemaphoreType.DMA,
            pltpu.SemaphoreType.DMA,
            pltpu.SemaphoreType.DMA,
            pltpu.SemaphoreType.DMA,
        ],
        compiler_params=pltpu.CompilerParams(
            use_tc_tiling_on_sc=True, needs_layout_passes=False
        ),
    )
    return gat(xT, tab2)


def kernel(x, seq_lengths, table):
    del seq_lengths
    xT = x.T.astype(jnp.int32)
    tT = table.T
    tailpair = table[NSTRIP * 128:, :].reshape(TAIL_V // 2, 2 * EMBED)
    outT = _sc_embed(xT, tT, tailpair)
    return jnp.transpose(outT, (2, 0, 1))


# speedup vs baseline: 1.2449x; 1.2449x over previous
import jax
import jax.numpy as jnp
from jax import lax
from jax.experimental import pallas as pl
from jax.experimental.pallas import tpu as pltpu
from jax.experimental.pallas import tpu_sc as plsc

VOCAB = 1000000
EMBED = 64
BATCH = 4096
SEQLEN = 50

NC = 2
NS = 16
LANES = 16
NW = NC * NS
BB = BATCH // NW
NGR = BB // LANES
ZROW = BB

NSTRIP = VOCAB // 128
SW = 256
NPAIR = NSTRIP // 2
PAIRS_MAIN = NPAIR - (NPAIR % NW)
PER_W_PAIRS = PAIRS_MAIN // NW
TAIL_PAIRS = NPAIR % NW
TAIL_V = VOCAB - NSTRIP * 128


def _iota16():
    return lax.iota(jnp.int32, LANES)


def _full16(v):
    return jnp.full((LANES,), v, jnp.int32)


def _transpose_strip(sb, tb, ngroups):
    def grp(vg, carry):
        vlv = vg * LANES + _iota16()
        trow = lax.shift_right_logical(vlv, 1)
        tcolb = lax.shift_left(jnp.bitwise_and(vlv, _full16(1)), _full16(6))
        for fb in range(EMBED // LANES):
            fbase = _full16(fb * LANES)

            @plsc.parallel_loop(0, LANES, unroll=8)
            def k_body(k):
                rot = jnp.bitwise_and(
                    _iota16() + _full16(k), _full16(LANES - 1)
                )
                fvec = fbase + rot
                val = plsc.load_gather(sb, [fvec, vlv])
                plsc.store_scatter(tb, [trow, tcolb + fvec], val)

        return carry

    lax.fori_loop(0, ngroups, grp, 0)


def _format_body(tT_hbm, tailp_hbm, out_hbm,
                 sb0, sb1, tb0, tb1, gs0, gs1, os0, os1):
    wid = lax.axis_index("s") * NC + lax.axis_index("c")
    bufs = ((sb0, tb0, gs0, os0), (sb1, tb1, gs1, os1))

    def pair_of(j):
        return wid + j * NW

    def fire_in(j, sb, gs):
        col = pl.multiple_of(pair_of(j) * SW, SW)
        pltpu.async_copy(tT_hbm.at[:, pl.ds(col, SW)], sb, gs)

    def drain_in(sb, gs):
        pltpu.make_async_copy(
            tT_hbm.at[:, pl.ds(0, SW)], sb, gs
        ).wait()

    def fire_out(j, tb, os):
        row = pl.multiple_of(pair_of(j) * (SW // 2), SW // 2)
        pltpu.async_copy(tb, out_hbm.at[pl.ds(row, SW // 2)], os)

    def drain_out(tb, os):
        pltpu.make_async_copy(
            tb, out_hbm.at[pl.ds(0, SW // 2)], os
        ).wait()

    for par in (0, 1):
        sb, tb, gs, os = bufs[par]
        fire_in(jnp.int32(par), sb, gs)
    for par in (0, 1):
        sb, tb, gs, os = bufs[par]
        drain_in(sb, gs)
        _transpose_strip(sb, tb, SW // LANES)
        fire_out(jnp.int32(par), tb, os)
        fire_in(jnp.int32(par + 2), sb, gs)

    def loop_body(i, carry):
        for par in (0, 1):
            sb, tb, gs, os = bufs[par]
            j = 2 * i + par
            drain_in(sb, gs)
            drain_out(tb, os)
            _transpose_strip(sb, tb, SW // LANES)
            fire_out(j, tb, os)
            fire_in(jnp.minimum(j + 2, PER_W_PAIRS - 1), sb, gs)
        return carry

    lax.fori_loop(1, PER_W_PAIRS // 2, loop_body, 0)

    for par in (0, 1):
        sb, tb, gs, os = bufs[par]
        drain_in(sb, gs)
        drain_out(tb, os)

    @pl.when(wid < TAIL_PAIRS)
    def _tail_full():
        p = PAIRS_MAIN + wid
        col = pl.multiple_of(p * SW, SW)
        pltpu.sync_copy(tT_hbm.at[:, pl.ds(col, SW)], sb0)
        _transpose_strip(sb0, tb0, SW // LANES)
        pltpu.sync_copy(tb0, out_hbm.at[pl.ds(p * (SW // 2), SW // 2)])

    @pl.when(wid == TAIL_PAIRS)
    def _tail_half():
        pltpu.sync_copy(tailp_hbm, tb0.at[pl.ds(0, TAIL_V // 2)])
        pltpu.sync_copy(
            tb0.at[pl.ds(0, TAIL_V // 2)],
            out_hbm.at[pl.ds(NSTRIP * 64, TAIL_V // 2)],
        )


def _gather_body(xT_hbm, tab_hbm, out_hbm,
                 iv_v, idx2_0, idx2_1, rz0, rz1, tb0, tb1,
                 gs0, gs1, os0, os1):
    wid = lax.axis_index("s") * NC + lax.axis_index("c")
    bcol = pl.multiple_of(wid * BB, BB)

    pltpu.sync_copy(xT_hbm.at[:, pl.ds(bcol, BB)], iv_v)

    for c in range(NGR):
        rz0[ZROW, pl.ds(c * LANES, LANES)] = jnp.zeros((LANES,), jnp.float32)
        rz1[ZROW, pl.ds(c * LANES, LANES)] = jnp.zeros((LANES,), jnp.float32)

    def prep(l, idx2_v):
        for g in range(NGR):
            ivg = plsc.load_gather(
                iv_v, [_full16(l), g * LANES + _iota16()]
            )
            idx2_v[pl.ds(g * LANES, LANES)] = lax.shift_right_logical(ivg, 1)

    def fire(idx2_v, rz, sem):
        pltpu.async_copy(tab_hbm.at[idx2_v], rz.at[pl.ds(0, BB)], sem)

    def drain_g(rz, sem):
        pltpu.make_async_copy(
            tab_hbm.at[pl.ds(0, BB)], rz.at[pl.ds(0, BB)], sem
        ).wait()

    def transpose(l, rz, tb):
        def bgrp_body(bg, carry):
            ivg = plsc.load_gather(
                iv_v, [_full16(l), bg * LANES + _iota16()]
            )
            half = lax.shift_left(
                jnp.bitwise_and(ivg, _full16(1)), _full16(6)
            )
            rowv = jnp.where(
                ivg == 0, _full16(ZROW), bg * LANES + _iota16()
            )
            colv = bg * LANES + _iota16()
            for fb in range(EMBED // LANES):
                fbase = _full16(fb * LANES)

                @plsc.parallel_loop(0, LANES, unroll=8)
                def k_body(k):
                    fvec = fbase + jnp.bitwise_and(
                        _iota16() + _full16(k), _full16(LANES - 1)
                    )
                    val = plsc.load_gather(rz, [rowv, half + fvec])
                    plsc.store_scatter(tb, [fvec, colv], val)

            return carry

        lax.fori_loop(0, NGR, bgrp_body, 0)

    def outdma(l, tb, sem):
        pltpu.async_copy(tb, out_hbm.at[l, :, pl.ds(bcol, BB)], sem)

    def drain_o(tb, sem):
        pltpu.make_async_copy(
            tb, out_hbm.at[0, :, pl.ds(bcol, BB)], sem
        ).wait()

    bufs = ((idx2_0, rz0, tb0, gs0, os0), (idx2_1, rz1, tb1, gs1, os1))

    for par in (0, 1):
        idx2_v, rz, tb, gs, os = bufs[par]
        prep(jnp.int32(par), idx2_v)
        fire(idx2_v, rz, gs)

    for par in (0, 1):
        idx2_v, rz, tb, gs, os = bufs[par]
        l = jnp.int32(par)
        drain_g(rz, gs)
        transpose(l, rz, tb)
        outdma(l, tb, os)
        prep(l + 2, idx2_v)
        fire(idx2_v, rz, gs)

    def loop_body(i, carry):
        for par in (0, 1):
            idx2_v, rz, tb, gs, os = bufs[par]
            l = 2 * i + par
            drain_g(rz, gs)
            drain_o(tb, os)
            transpose(l, rz, tb)
            outdma(l, tb, os)
            prep(jnp.minimum(l + 2, SEQLEN - 1), idx2_v)
            fire(idx2_v, rz, gs)
        return carry

    lax.fori_loop(1, SEQLEN // 2, loop_body, 0)

    for par in (0, 1):
        idx2_v, rz, tb, gs, os = bufs[par]
        drain_g(rz, gs)
        drain_o(tb, os)


@jax.jit
def _sc_embed(xT, tT, tailpair):
    fmt = pl.kernel(
        _format_body,
        out_type=jax.ShapeDtypeStruct((VOCAB // 2, 2 * EMBED), jnp.float32),
        mesh=plsc.VectorSubcoreMesh(
            core_axis_name="c", subcore_axis_name="s"
        ),
        scratch_types=[
            pltpu.VMEM((EMBED, SW), jnp.float32),
            pltpu.VMEM((EMBED, SW), jnp.float32),
            pltpu.VMEM((SW // 2, 128), jnp.float32),
            pltpu.VMEM((SW // 2, 128), jnp.float32),
            pltpu.SemaphoreType.DMA,
            pltpu.SemaphoreType.DMA,
            pltpu.SemaphoreType.DMA,
            pltpu.SemaphoreType.DMA,
        ],
        compiler_params=pltpu.CompilerParams(
            use_tc_tiling_on_sc=True, needs_layout_passes=False
        ),
    )
    tab2 = fmt(tT, tailpair)

    gat = pl.kernel(
        _gather_body,
        out_type=jax.ShapeDtypeStruct((SEQLEN, EMBED, BATCH), jnp.float32),
        mesh=plsc.VectorSubcoreMesh(
            core_axis_name="c", subcore_axis_name="s"
        ),
        scratch_types=[
            pltpu.VMEM((SEQLEN, BB), jnp.int32),
            pltpu.VMEM((BB,), jnp.int32),
            pltpu.VMEM((BB,), jnp.int32),
            pltpu.VMEM((BB + 8, BB), jnp.float32),
            pltpu.VMEM((BB + 8, BB), jnp.float32),
            pltpu.VMEM((EMBED, BB), jnp.float32),
            pltpu.VMEM((EMBED, BB), jnp.float32),
            pltpu.SemaphoreType.DMA,
            pltpu.SemaphoreType.DMA,
            pltpu.SemaphoreType.DMA,
            pltpu.SemaphoreType.DMA,
        ],
        compiler_params=pltpu.CompilerParams(
            use_tc_tiling_on_sc=True, needs_layout_passes=False
        ),
    )
    return gat(xT, tab2)


def kernel(x, seq_lengths, table):
    del seq_lengths
    xT = x.T.astype(jnp.int32)
    tT = table.T
    tailpair = table[NSTRIP * 128:, :].reshape(TAIL_V // 2, 2 * EMBED)
    outT = _sc_embed(xT, tT, tailpair)
    return jnp.transpose(outT, (2, 0, 1))


# speedup vs baseline: 1.5830x; 1.2715x over previous
import jax
import jax.numpy as jnp
from jax import lax
from jax.experimental import pallas as pl
from jax.experimental.pallas import tpu as pltpu
from jax.experimental.pallas import tpu_sc as plsc

VOCAB = 1000000
EMBED = 64
BATCH = 4096
SEQLEN = 50

NC = 2
NS = 16
LANES = 16
NW = NC * NS
BB = BATCH // NW
NGR = BB // LANES
ZROW = BB

NSTRIP = VOCAB // 128
SW = 256
NPAIR = NSTRIP // 2
PAIRS_MAIN = NPAIR - (NPAIR % NW)
PER_W_PAIRS = PAIRS_MAIN // NW
TAIL_PAIRS = NPAIR % NW
TAIL_V = VOCAB - NSTRIP * 128


def _iota16():
    return lax.iota(jnp.int32, LANES)


def _full16(v):
    return jnp.full((LANES,), v, jnp.int32)


def _transpose_strip(sb, tb, ngroups):
    def grp(vg, carry):
        vlv = vg * LANES + _iota16()
        trow = lax.shift_right_logical(vlv, 1)
        tcolb = lax.shift_left(jnp.bitwise_and(vlv, _full16(1)), _full16(6))
        for fb in range(EMBED // LANES):
            fbase = _full16(fb * LANES)

            @plsc.parallel_loop(0, LANES, unroll=16)
            def k_body(k):
                rot = jnp.bitwise_and(
                    _iota16() + _full16(k), _full16(LANES - 1)
                )
                fvec = fbase + rot
                val = plsc.load_gather(sb, [fvec, vlv])
                plsc.store_scatter(tb, [trow, tcolb + fvec], val)

        return carry

    lax.fori_loop(0, ngroups, grp, 0)


def _format_body(tT_hbm, tailp_hbm, out_hbm,
                 sb0, sb1, tb0, tb1, gs0, gs1, os0, os1):
    wid = lax.axis_index("s") * NC + lax.axis_index("c")
    bufs = ((sb0, tb0, gs0, os0), (sb1, tb1, gs1, os1))

    def pair_of(j):
        return wid + j * NW

    def fire_in(j, sb, gs):
        col = pl.multiple_of(pair_of(j) * SW, SW)
        pltpu.async_copy(tT_hbm.at[:, pl.ds(col, SW)], sb, gs)

    def drain_in(sb, gs):
        pltpu.make_async_copy(
            tT_hbm.at[:, pl.ds(0, SW)], sb, gs
        ).wait()

    def fire_out(j, tb, os):
        row = pl.multiple_of(pair_of(j) * (SW // 2), SW // 2)
        pltpu.async_copy(tb, out_hbm.at[pl.ds(row, SW // 2)], os)

    def drain_out(tb, os):
        pltpu.make_async_copy(
            tb, out_hbm.at[pl.ds(0, SW // 2)], os
        ).wait()

    for par in (0, 1):
        sb, tb, gs, os = bufs[par]
        fire_in(jnp.int32(par), sb, gs)
    for par in (0, 1):
        sb, tb, gs, os = bufs[par]
        drain_in(sb, gs)
        _transpose_strip(sb, tb, SW // LANES)
        fire_out(jnp.int32(par), tb, os)
        fire_in(jnp.int32(par + 2), sb, gs)

    def loop_body(i, carry):
        for par in (0, 1):
            sb, tb, gs, os = bufs[par]
            j = 2 * i + par
            drain_in(sb, gs)
            drain_out(tb, os)
            _transpose_strip(sb, tb, SW // LANES)
            fire_out(j, tb, os)
            fire_in(jnp.minimum(j + 2, PER_W_PAIRS - 1), sb, gs)
        return carry

    lax.fori_loop(1, PER_W_PAIRS // 2, loop_body, 0)

    for par in (0, 1):
        sb, tb, gs, os = bufs[par]
        drain_in(sb, gs)
        drain_out(tb, os)

    @pl.when(wid < TAIL_PAIRS)
    def _tail_full():
        p = PAIRS_MAIN + wid
        col = pl.multiple_of(p * SW, SW)
        pltpu.sync_copy(tT_hbm.at[:, pl.ds(col, SW)], sb0)
        _transpose_strip(sb0, tb0, SW // LANES)
        pltpu.sync_copy(tb0, out_hbm.at[pl.ds(p * (SW // 2), SW // 2)])

    @pl.when(wid == TAIL_PAIRS)
    def _tail_half():
        pltpu.sync_copy(tailp_hbm, tb0.at[pl.ds(0, TAIL_V // 2)])
        pltpu.sync_copy(
            tb0.at[pl.ds(0, TAIL_V // 2)],
            out_hbm.at[pl.ds(NSTRIP * 64, TAIL_V // 2)],
        )


def _gather_body(xT_hbm, tab_hbm, out_hbm,
                 iv_v, idx2_0, idx2_1, rz0, rz1, tb0, tb1,
                 gs0, gs1, os0, os1):
    wid = lax.axis_index("s") * NC + lax.axis_index("c")
    bcol = pl.multiple_of(wid * BB, BB)

    pltpu.sync_copy(xT_hbm.at[:, pl.ds(bcol, BB)], iv_v)

    for c in range(NGR):
        rz0[ZROW, pl.ds(c * LANES, LANES)] = jnp.zeros((LANES,), jnp.float32)
        rz1[ZROW, pl.ds(c * LANES, LANES)] = jnp.zeros((LANES,), jnp.float32)

    def prep(l, idx2_v):
        for g in range(NGR):
            ivg = plsc.load_gather(
                iv_v, [_full16(l), g * LANES + _iota16()]
            )
            idx2_v[pl.ds(g * LANES, LANES)] = lax.shift_right_logical(ivg, 1)

    def fire(idx2_v, rz, sem):
        pltpu.async_copy(tab_hbm.at[idx2_v], rz.at[pl.ds(0, BB)], sem)

    def drain_g(rz, sem):
        pltpu.make_async_copy(
            tab_hbm.at[pl.ds(0, BB)], rz.at[pl.ds(0, BB)], sem
        ).wait()

    def transpose(l, rz, tb):
        def bgrp_body(bg, carry):
            ivg = plsc.load_gather(
                iv_v, [_full16(l), bg * LANES + _iota16()]
            )
            half = lax.shift_left(
                jnp.bitwise_and(ivg, _full16(1)), _full16(6)
            )
            rowv = jnp.where(
                ivg == 0, _full16(ZROW), bg * LANES + _iota16()
            )
            colv = bg * LANES + _iota16()
            for fb in range(EMBED // LANES):
                fbase = _full16(fb * LANES)

                @plsc.parallel_loop(0, LANES, unroll=16)
                def k_body(k):
                    fvec = fbase + jnp.bitwise_and(
                        _iota16() + _full16(k), _full16(LANES - 1)
                    )
                    val = plsc.load_gather(rz, [rowv, half + fvec])
                    plsc.store_scatter(tb, [fvec, colv], val)

            return carry

        lax.fori_loop(0, NGR, bgrp_body, 0)

    def outdma(l, tb, sem):
        pltpu.async_copy(tb, out_hbm.at[l, :, pl.ds(bcol, BB)], sem)

    def drain_o(tb, sem):
        pltpu.make_async_copy(
            tb, out_hbm.at[0, :, pl.ds(bcol, BB)], sem
        ).wait()

    bufs = ((idx2_0, rz0, tb0, gs0, os0), (idx2_1, rz1, tb1, gs1, os1))

    for par in (0, 1):
        idx2_v, rz, tb, gs, os = bufs[par]
        prep(jnp.int32(par), idx2_v)
        fire(idx2_v, rz, gs)

    for par in (0, 1):
        idx2_v, rz, tb, gs, os = bufs[par]
        l = jnp.int32(par)
        drain_g(rz, gs)
        transpose(l, rz, tb)
        outdma(l, tb, os)
        prep(l + 2, idx2_v)
        fire(idx2_v, rz, gs)

    def loop_body(i, carry):
        for par in (0, 1):
            idx2_v, rz, tb, gs, os = bufs[par]
            l = 2 * i + par
            drain_g(rz, gs)
            drain_o(tb, os)
            transpose(l, rz, tb)
            outdma(l, tb, os)
            prep(jnp.minimum(l + 2, SEQLEN - 1), idx2_v)
            fire(idx2_v, rz, gs)
        return carry

    lax.fori_loop(1, SEQLEN // 2, loop_body, 0)

    for par in (0, 1):
        idx2_v, rz, tb, gs, os = bufs[par]
        drain_g(rz, gs)
        drain_o(tb, os)


@jax.jit
def _sc_embed(xT, tT, tailpair):
    fmt = pl.kernel(
        _format_body,
        out_type=jax.ShapeDtypeStruct((VOCAB // 2, 2 * EMBED), jnp.float32),
        mesh=plsc.VectorSubcoreMesh(
            core_axis_name="c", subcore_axis_name="s"
        ),
        scratch_types=[
            pltpu.VMEM((EMBED, SW), jnp.float32),
            pltpu.VMEM((EMBED, SW), jnp.float32),
            pltpu.VMEM((SW // 2, 128), jnp.float32),
            pltpu.VMEM((SW // 2, 128), jnp.float32),
            pltpu.SemaphoreType.DMA,
            pltpu.SemaphoreType.DMA,
            pltpu.SemaphoreType.DMA,
            pltpu.SemaphoreType.DMA,
        ],
        compiler_params=pltpu.CompilerParams(
            use_tc_tiling_on_sc=True, needs_layout_passes=False
        ),
    )
    tab2 = fmt(tT, tailpair)

    gat = pl.kernel(
        _gather_body,
        out_type=jax.ShapeDtypeStruct((SEQLEN, EMBED, BATCH), jnp.float32),
        mesh=plsc.VectorSubcoreMesh(
            core_axis_name="c", subcore_axis_name="s"
        ),
        scratch_types=[
            pltpu.VMEM((SEQLEN, BB), jnp.int32),
            pltpu.VMEM((BB,), jnp.int32),
            pltpu.VMEM((BB,), jnp.int32),
            pltpu.VMEM((BB + 8, BB), jnp.float32),
            pltpu.VMEM((BB + 8, BB), jnp.float32),
            pltpu.VMEM((EMBED, BB), jnp.float32),
            pltpu.VMEM((EMBED, BB), jnp.float32),
            pltpu.SemaphoreType.DMA,
            pltpu.SemaphoreType.DMA,
            pltpu.SemaphoreType.DMA,
            pltpu.SemaphoreType.DMA,
        ],
        compiler_params=pltpu.CompilerParams(
            use_tc_tiling_on_sc=True, needs_layout_passes=False
        ),
    )
    return gat(xT, tab2)


def kernel(x, seq_lengths, table):
    del seq_lengths
    xT = x.T.astype(jnp.int32)
    tT = table.T
    tailpair = table[NSTRIP * 128:, :].reshape(TAIL_V // 2, 2 * EMBED)
    outT = _sc_embed(xT, tT, tailpair)
    return jnp.transpose(outT, (2, 0, 1))
